# baseline (speedup 1.0000x reference)
import functools

import jax
import jax.numpy as jnp
from jax import lax
from jax.experimental import pallas as pl
from jax.experimental.pallas import tpu as pltpu
from jax.experimental.pallas import tpu_sc as plsc

E = 16
D = 1024
F = 768
T = 4096
R = 512
G = (2 * T) // R + E
S = G * R
C = 512

NC = 2
NS = 16
NW = NC * NS
TOK_W = T // NW
CH = 32


def _router_body(x_ref, wg_ref, meta_ref, blk_ref, slots_ref):
    x = x_ref[...]
    wg = wg_ref[...]
    logits = lax.dot_general(x, wg, (((1,), (1,)), ((), ())),
                             preferred_element_type=jnp.float32)
    lane = lax.broadcasted_iota(jnp.int32, (T, E), 1)
    m1 = jnp.max(logits, axis=1, keepdims=True)
    e0 = jnp.min(jnp.where(logits == m1, lane, E), axis=1, keepdims=True)
    logits2 = jnp.where(lane == e0, -jnp.inf, logits)
    m2 = jnp.max(logits2, axis=1, keepdims=True)
    e1 = jnp.min(jnp.where(logits2 == m2, lane, E), axis=1, keepdims=True)
    w0 = 1.0 / (1.0 + jnp.exp(m2 - m1))
    w1 = 1.0 - w0

    oh0 = (lane == e0).astype(jnp.float32)
    oh1 = (lane == e1).astype(jnp.float32)

    a = jnp.concatenate([oh0, oh1], axis=0)
    k = 1
    while k < 2 * T:
        a = a + jnp.concatenate(
            [jnp.zeros((k, E), jnp.float32), a[:2 * T - k, :]], axis=0)
        k *= 2
    counts = a[2 * T - 1:2 * T, :]
    rank0 = a[:T, :]
    rank1 = a[T:, :]

    nblk = jnp.ceil(counts / R)
    er = lax.broadcasted_iota(jnp.int32, (E, E), 0)
    ec = lax.broadcasted_iota(jnp.int32, (E, E), 1)
    m_lt = (er < ec).astype(jnp.float32)
    m_le = (er <= ec).astype(jnp.float32)
    pad_base = lax.dot_general(nblk, m_lt, (((1,), (0,)), ((), ())),
                               preferred_element_type=jnp.float32) * R
    cum_incl = lax.dot_general(nblk, m_le, (((1,), (0,)), ((), ())),
                               preferred_element_type=jnp.float32)
    total_blocks = cum_incl[:, E - 1:E]

    slot0 = jnp.sum(oh0 * (rank0 - 1.0 + pad_base), axis=1, keepdims=True)
    slot1 = jnp.sum(oh1 * (rank1 - 1.0 + pad_base), axis=1, keepdims=True)
    slots_ref[...] = jnp.concatenate([slot0, slot1],
                                     axis=0).astype(jnp.int32)

    lane128 = lax.broadcasted_iota(jnp.int32, (T, 128), 1)
    meta = jnp.where(lane128 == 0, slot0,
                     jnp.where(lane128 == 1, slot1,
                               jnp.where(lane128 < 32, w0, w1)))
    meta_ref[...] = meta

    gidx = lax.broadcasted_iota(jnp.int32, (96, E), 0).astype(jnp.float32)
    act = (counts > 0.0).astype(jnp.float32)
    cum_excl = cum_incl - nblk
    arank = lax.dot_general(act, m_lt, (((1,), (0,)), ((), ())),
                            preferred_element_type=jnp.float32)
    bexp = jnp.sum((gidx >= cum_incl).astype(jnp.float32), axis=1,
                   keepdims=True)
    bexp = jnp.minimum(bexp, float(E - 1))
    is_first = (gidx == cum_excl) * act
    chg = jnp.sum(is_first, axis=1, keepdims=True)
    run_idx = jnp.sum((gidx >= cum_excl) * act, axis=1,
                      keepdims=True) - 1.0
    slot = run_idx - 3.0 * jnp.floor(run_idx / 3.0)
    eids = lax.broadcasted_iota(jnp.int32, (96, E), 1).astype(jnp.float32)
    sel1 = (arank == run_idx + 1.0) * act
    er1 = jnp.sum(sel1 * eids, axis=1, keepdims=True)
    v1 = jnp.minimum(jnp.sum(sel1, axis=1, keepdims=True), 1.0)
    sel2 = (arank == run_idx + 2.0) * act
    er2 = jnp.sum(sel2 * eids, axis=1, keepdims=True)
    v2 = jnp.minimum(jnp.sum(sel2, axis=1, keepdims=True), 1.0)
    packed = (bexp + slot * 32.0 + chg * 128.0 + er1 * 256.0
              + v1 * 8192.0 + er2 * 16384.0 + v2 * 524288.0)
    row = lax.broadcasted_iota(jnp.int32, (96, 1), 0)
    vals = jnp.where(row == G, total_blocks, packed)
    blk_ref[...] = vals.astype(jnp.int32)


NCH = TOK_W // CH


def _dispatch_body(x_hbm, slots2_hbm, xs_hbm, idx_v, rows_v, semL, semS):
    wid = lax.axis_index("s") * NC + lax.axis_index("c")
    base = wid * TOK_W
    rb = wid * NCH

    pltpu.sync_copy(slots2_hbm.at[pl.ds(rb, NCH)], idx_v.at[pl.ds(0, NCH)])
    pltpu.sync_copy(slots2_hbm.at[pl.ds(T // CH + rb, NCH)],
                    idx_v.at[pl.ds(NCH, NCH)])

    def load(c, b):
        return pltpu.make_async_copy(x_hbm.at[pl.ds(base + c * CH, CH)],
                                     rows_v.at[b], semL.at[b])

    def scat(c, b, k):
        return pltpu.make_async_copy(rows_v.at[b],
                                     xs_hbm.at[idx_v.at[k * NCH + c]],
                                     semS.at[b])

    load(0, 0).start()
    for c in range(NCH):
        b = c % 2
        load(c, b).wait()
        if c + 1 < NCH:
            if c >= 1:
                scat(c - 1, 1 - b, 0).wait()
                scat(c - 1, 1 - b, 1).wait()
            load(c + 1, 1 - b).start()
        scat(c, b, 0).start()
        scat(c, b, 1).start()
    for c in (NCH - 2, NCH - 1):
        scat(c, c % 2, 0).wait()
        scat(c, c % 2, 1).wait()


def _ffn_body(info_ref, xs_ref, wgu_hbm, wd_hbm, ys_ref,
              wgu_buf, wd_buf, sem_gu, sem_d):
    g = pl.program_id(0)
    nb = info_ref[G, 0]
    info = info_ref[g, 0]
    eid = lax.rem(info, 32)
    slot = lax.rem(info // 32, 4)
    chg = lax.rem(info // 128, 2)
    er1 = lax.rem(info // 256, 32)
    v1 = lax.rem(info // 8192, 2)
    er2 = lax.rem(info // 16384, 32)
    v2 = lax.rem(info // 524288, 2)

    def start_fetch(e, s):
        pltpu.make_async_copy(wgu_hbm.at[e], wgu_buf.at[s],
                              sem_gu.at[s]).start()
        pltpu.make_async_copy(wd_hbm.at[e], wd_buf.at[s],
                              sem_d.at[s]).start()

    @pl.when(g == 0)
    def _():
        start_fetch(eid, slot)

        @pl.when(v1 == 1)
        def _():
            start_fetch(er1, lax.rem(slot + 1, 3))

    @pl.when((g < nb) & (chg == 1))
    def _():
        @pl.when(v2 == 1)
        def _():
            start_fetch(er2, lax.rem(slot + 2, 3))

        pltpu.make_async_copy(wgu_hbm.at[eid], wgu_buf.at[slot],
                              sem_gu.at[slot]).wait()
        pltpu.make_async_copy(wd_hbm.at[eid], wd_buf.at[slot],
                              sem_d.at[slot]).wait()

    @pl.when(g < nb)
    def _():
        x = xs_ref[...]
        wgu = wgu_buf[slot]
        gu = lax.dot_general(x, wgu, (((1,), (1,)), ((), ())),
                             preferred_element_type=jnp.float32)
        gate = gu[:, :F]
        up = gu[:, F:]
        h = gate * (1.0 / (1.0 + jnp.exp(-gate))) * up
        wd = wd_buf[slot]
        ys_ref[...] = lax.dot_general(h, wd, (((1,), (1,)), ((), ())),
                                      preferred_element_type=jnp.float32)


CHC = 16
NCC = TOK_W // CHC


def _combine_body(ys_hbm, slots3_hbm, meta_hbm, out_hbm, idx_v, wr_v,
                  r0_v, r1_v, o_v, semG0, semG1, semO):
    wid = lax.axis_index("s") * NC + lax.axis_index("c")
    base = wid * TOK_W
    rb = wid * NCC
    L = 16

    pltpu.sync_copy(slots3_hbm.at[pl.ds(rb, NCC)], idx_v.at[pl.ds(0, NCC)])
    pltpu.sync_copy(slots3_hbm.at[pl.ds(T // CHC + rb, NCC)],
                    idx_v.at[pl.ds(NCC, NCC)])
    pltpu.sync_copy(meta_hbm.at[pl.ds(base, TOK_W)], wr_v)

    def gath(c, b, k, dst, sem):
        return pltpu.make_async_copy(ys_hbm.at[idx_v.at[k * NCC + c]],
                                     dst.at[b], sem.at[b])

    def store(c, b):
        return pltpu.make_async_copy(o_v.at[b],
                                     out_hbm.at[pl.ds(base + c * CHC, CHC)],
                                     semO.at[b])

    gath(0, 0, 0, r0_v, semG0).start()
    gath(0, 0, 1, r1_v, semG1).start()
    for c in range(NCC):
        b = c % 2
        if c + 1 < NCC:
            gath(c + 1, 1 - b, 0, r0_v, semG0).start()
            gath(c + 1, 1 - b, 1, r1_v, semG1).start()
        gath(c, b, 0, r0_v, semG0).wait()
        gath(c, b, 1, r1_v, semG1).wait()
        if c >= 2:
            store(c - 2, b).wait()

        def tok(r, carry):
            w0b = wr_v[c * CHC + r, pl.ds(16, L)]
            w1b = wr_v[c * CHC + r, pl.ds(32, L)]
            for s in range(D // L):
                sl = pl.ds(s * L, L)
                o_v[b, r, sl] = w0b * r0_v[b, r, sl] + w1b * r1_v[b, r, sl]
            return carry

        lax.fori_loop(0, CHC, tok, 0)
        store(c, b).start()
    for c in (NCC - 2, NCC - 1):
        store(c, c % 2).wait()


def kernel(hidden_states, Wg, Wgu, Wd):
    x = hidden_states

    meta, blk, slots_a = pl.pallas_call(
        _router_body,
        out_shape=[
            jax.ShapeDtypeStruct((T, 128), jnp.float32),
            jax.ShapeDtypeStruct((96, 1), jnp.int32),
            jax.ShapeDtypeStruct((2 * T, 1), jnp.int32),
        ],
    )(x, Wg)

    slots2 = slots_a.reshape(2 * T // CH, CH)
    slots3 = slots_a.reshape(2 * T // CHC, CHC)
    prefetch = blk

    mesh = plsc.VectorSubcoreMesh(core_axis_name="c", subcore_axis_name="s")

    xs = pl.kernel(
        _dispatch_body,
        out_type=jax.ShapeDtypeStruct((S, D), jnp.float32),
        mesh=mesh,
        scratch_types=[
            pltpu.VMEM((2 * NCH, CH), jnp.int32),
            pltpu.VMEM((2, CH, D), jnp.float32),
            pltpu.SemaphoreType.DMA((2,)),
            pltpu.SemaphoreType.DMA((2,)),
        ],
    )(x, slots2)

    grid_spec = pltpu.PrefetchScalarGridSpec(
        num_scalar_prefetch=1,
        grid=(G,),
        in_specs=[
            pl.BlockSpec((R, D),
                         lambda g, pref: (jnp.minimum(g, pref[G, 0] - 1), 0)),
            pl.BlockSpec(memory_space=pl.ANY),
            pl.BlockSpec(memory_space=pl.ANY),
        ],
        out_specs=pl.BlockSpec(
            (R, D), lambda g, pref: (jnp.minimum(g, pref[G, 0] - 1), 0)),
        scratch_shapes=[
            pltpu.VMEM((3, 2 * F, D), jnp.float32),
            pltpu.VMEM((3, D, F), jnp.float32),
            pltpu.SemaphoreType.DMA((3,)),
            pltpu.SemaphoreType.DMA((3,)),
        ],
    )
    ys = pl.pallas_call(
        _ffn_body,
        grid_spec=grid_spec,
        out_shape=jax.ShapeDtypeStruct((S, D), jnp.float32),
        compiler_params=pltpu.CompilerParams(
            dimension_semantics=("arbitrary",)),
    )(prefetch, xs, Wgu, Wd)

    out = pl.kernel(
        _combine_body,
        out_type=jax.ShapeDtypeStruct((T, D), jnp.float32),
        mesh=mesh,
        scratch_types=[
            pltpu.VMEM((2 * NCC, CHC), jnp.int32),
            pltpu.VMEM((TOK_W, 128), jnp.float32),
            pltpu.VMEM((2, CHC, D), jnp.float32),
            pltpu.VMEM((2, CHC, D), jnp.float32),
            pltpu.VMEM((2, CHC, D), jnp.float32),
            pltpu.SemaphoreType.DMA((2,)),
            pltpu.SemaphoreType.DMA((2,)),
            pltpu.SemaphoreType.DMA((2,)),
        ],
    )(ys, slots3, meta)
    return out


# speedup vs baseline: 1.0386x; 1.0386x over previous
import functools

import jax
import jax.numpy as jnp
from jax import lax
from jax.experimental import pallas as pl
from jax.experimental.pallas import tpu as pltpu
from jax.experimental.pallas import tpu_sc as plsc

E = 16
D = 1024
F = 768
T = 4096
R = 768
G = (2 * T + R - 1) // R + E
S = G * R
C = 512

NC = 2
NS = 16
NW = NC * NS
TOK_W = T // NW
CH = 32


def _router_body(x_ref, wg_ref, meta_ref, blk_ref, slots_ref):
    x = x_ref[...]
    wg = wg_ref[...]
    logits = lax.dot_general(x, wg, (((1,), (1,)), ((), ())),
                             preferred_element_type=jnp.float32)
    lane = lax.broadcasted_iota(jnp.int32, (T, E), 1)
    m1 = jnp.max(logits, axis=1, keepdims=True)
    e0 = jnp.min(jnp.where(logits == m1, lane, E), axis=1, keepdims=True)
    logits2 = jnp.where(lane == e0, -jnp.inf, logits)
    m2 = jnp.max(logits2, axis=1, keepdims=True)
    e1 = jnp.min(jnp.where(logits2 == m2, lane, E), axis=1, keepdims=True)
    w0 = 1.0 / (1.0 + jnp.exp(m2 - m1))
    w1 = 1.0 - w0

    oh0 = (lane == e0).astype(jnp.float32)
    oh1 = (lane == e1).astype(jnp.float32)

    a = jnp.concatenate([oh0, oh1], axis=0)
    k = 1
    while k < 2 * T:
        a = a + jnp.concatenate(
            [jnp.zeros((k, E), jnp.float32), a[:2 * T - k, :]], axis=0)
        k *= 2
    counts = a[2 * T - 1:2 * T, :]
    rank0 = a[:T, :]
    rank1 = a[T:, :]

    nblk = jnp.ceil(counts / R)
    er = lax.broadcasted_iota(jnp.int32, (E, E), 0)
    ec = lax.broadcasted_iota(jnp.int32, (E, E), 1)
    m_lt = (er < ec).astype(jnp.float32)
    m_le = (er <= ec).astype(jnp.float32)
    pad_base = lax.dot_general(nblk, m_lt, (((1,), (0,)), ((), ())),
                               preferred_element_type=jnp.float32) * R
    cum_incl = lax.dot_general(nblk, m_le, (((1,), (0,)), ((), ())),
                               preferred_element_type=jnp.float32)
    total_blocks = cum_incl[:, E - 1:E]

    slot0 = jnp.sum(oh0 * (rank0 - 1.0 + pad_base), axis=1, keepdims=True)
    slot1 = jnp.sum(oh1 * (rank1 - 1.0 + pad_base), axis=1, keepdims=True)
    slots_ref[...] = jnp.concatenate([slot0, slot1],
                                     axis=0).astype(jnp.int32)

    lane128 = lax.broadcasted_iota(jnp.int32, (T, 128), 1)
    meta = jnp.where(lane128 == 0, slot0,
                     jnp.where(lane128 == 1, slot1,
                               jnp.where(lane128 < 32, w0, w1)))
    meta_ref[...] = meta

    gidx = lax.broadcasted_iota(jnp.int32, (96, E), 0).astype(jnp.float32)
    act = (counts > 0.0).astype(jnp.float32)
    cum_excl = cum_incl - nblk
    arank = lax.dot_general(act, m_lt, (((1,), (0,)), ((), ())),
                            preferred_element_type=jnp.float32)
    bexp = jnp.sum((gidx >= cum_incl).astype(jnp.float32), axis=1,
                   keepdims=True)
    bexp = jnp.minimum(bexp, float(E - 1))
    is_first = (gidx == cum_excl) * act
    chg = jnp.sum(is_first, axis=1, keepdims=True)
    run_idx = jnp.sum((gidx >= cum_excl) * act, axis=1,
                      keepdims=True) - 1.0
    slot = run_idx - 3.0 * jnp.floor(run_idx / 3.0)
    eids = lax.broadcasted_iota(jnp.int32, (96, E), 1).astype(jnp.float32)
    sel1 = (arank == run_idx + 1.0) * act
    er1 = jnp.sum(sel1 * eids, axis=1, keepdims=True)
    v1 = jnp.minimum(jnp.sum(sel1, axis=1, keepdims=True), 1.0)
    sel2 = (arank == run_idx + 2.0) * act
    er2 = jnp.sum(sel2 * eids, axis=1, keepdims=True)
    v2 = jnp.minimum(jnp.sum(sel2, axis=1, keepdims=True), 1.0)
    packed = (bexp + slot * 32.0 + chg * 128.0 + er1 * 256.0
              + v1 * 8192.0 + er2 * 16384.0 + v2 * 524288.0)
    row = lax.broadcasted_iota(jnp.int32, (96, 1), 0)
    vals = jnp.where(row == G, total_blocks, packed)
    blk_ref[...] = vals.astype(jnp.int32)


NCH = TOK_W // CH


def _dispatch_body(x_hbm, slots2_hbm, xs_hbm, idx_v, rows_v, semL, semS):
    wid = lax.axis_index("s") * NC + lax.axis_index("c")
    base = wid * TOK_W
    rb = wid * NCH

    pltpu.sync_copy(slots2_hbm.at[pl.ds(rb, NCH)], idx_v.at[pl.ds(0, NCH)])
    pltpu.sync_copy(slots2_hbm.at[pl.ds(T // CH + rb, NCH)],
                    idx_v.at[pl.ds(NCH, NCH)])

    def load(c, b):
        return pltpu.make_async_copy(x_hbm.at[pl.ds(base + c * CH, CH)],
                                     rows_v.at[b], semL.at[b])

    def scat(c, b, k):
        return pltpu.make_async_copy(rows_v.at[b],
                                     xs_hbm.at[idx_v.at[k * NCH + c]],
                                     semS.at[b])

    load(0, 0).start()
    for c in range(NCH):
        b = c % 2
        load(c, b).wait()
        if c + 1 < NCH:
            if c >= 1:
                scat(c - 1, 1 - b, 0).wait()
                scat(c - 1, 1 - b, 1).wait()
            load(c + 1, 1 - b).start()
        scat(c, b, 0).start()
        scat(c, b, 1).start()
    for c in (NCH - 2, NCH - 1):
        scat(c, c % 2, 0).wait()
        scat(c, c % 2, 1).wait()


def _ffn_body(info_ref, xs_ref, wgu_hbm, wd_hbm, ys_ref,
              wgu_buf, wd_buf, sem_gu, sem_d):
    g = pl.program_id(0)
    nb = info_ref[G, 0]
    info = info_ref[g, 0]
    eid = lax.rem(info, 32)
    slot = lax.rem(info // 32, 4)
    chg = lax.rem(info // 128, 2)
    er1 = lax.rem(info // 256, 32)
    v1 = lax.rem(info // 8192, 2)
    er2 = lax.rem(info // 16384, 32)
    v2 = lax.rem(info // 524288, 2)

    def start_fetch(e, s):
        pltpu.make_async_copy(wgu_hbm.at[e], wgu_buf.at[s],
                              sem_gu.at[s]).start()
        pltpu.make_async_copy(wd_hbm.at[e], wd_buf.at[s],
                              sem_d.at[s]).start()

    @pl.when(g == 0)
    def _():
        start_fetch(eid, slot)

        @pl.when(v1 == 1)
        def _():
            start_fetch(er1, lax.rem(slot + 1, 3))

    @pl.when((g < nb) & (chg == 1))
    def _():
        @pl.when(v2 == 1)
        def _():
            start_fetch(er2, lax.rem(slot + 2, 3))

        pltpu.make_async_copy(wgu_hbm.at[eid], wgu_buf.at[slot],
                              sem_gu.at[slot]).wait()
        pltpu.make_async_copy(wd_hbm.at[eid], wd_buf.at[slot],
                              sem_d.at[slot]).wait()

    @pl.when(g < nb)
    def _():
        x = xs_ref[...]
        wgu = wgu_buf[slot]
        gu = lax.dot_general(x, wgu, (((1,), (1,)), ((), ())),
                             preferred_element_type=jnp.float32)
        gate = gu[:, :F]
        up = gu[:, F:]
        h = gate * (1.0 / (1.0 + jnp.exp(-gate))) * up
        wd = wd_buf[slot]
        ys_ref[...] = lax.dot_general(h, wd, (((1,), (1,)), ((), ())),
                                      preferred_element_type=jnp.float32)


CHC = 16
NCC = TOK_W // CHC


def _combine_body(ys_hbm, slots3_hbm, meta_hbm, out_hbm, idx_v, wr_v,
                  r0_v, r1_v, o_v, semG0, semG1, semO):
    wid = lax.axis_index("s") * NC + lax.axis_index("c")
    base = wid * TOK_W
    rb = wid * NCC
    L = 16

    pltpu.sync_copy(slots3_hbm.at[pl.ds(rb, NCC)], idx_v.at[pl.ds(0, NCC)])
    pltpu.sync_copy(slots3_hbm.at[pl.ds(T // CHC + rb, NCC)],
                    idx_v.at[pl.ds(NCC, NCC)])
    pltpu.sync_copy(meta_hbm.at[pl.ds(base, TOK_W)], wr_v)

    def gath(c, b, k, dst, sem):
        return pltpu.make_async_copy(ys_hbm.at[idx_v.at[k * NCC + c]],
                                     dst.at[b], sem.at[b])

    def store(c, b):
        return pltpu.make_async_copy(o_v.at[b],
                                     out_hbm.at[pl.ds(base + c * CHC, CHC)],
                                     semO.at[b])

    gath(0, 0, 0, r0_v, semG0).start()
    gath(0, 0, 1, r1_v, semG1).start()
    for c in range(NCC):
        b = c % 2
        if c + 1 < NCC:
            gath(c + 1, 1 - b, 0, r0_v, semG0).start()
            gath(c + 1, 1 - b, 1, r1_v, semG1).start()
        gath(c, b, 0, r0_v, semG0).wait()
        gath(c, b, 1, r1_v, semG1).wait()
        if c >= 2:
            store(c - 2, b).wait()

        def tok(r, carry):
            w0b = wr_v[c * CHC + r, pl.ds(16, L)]
            w1b = wr_v[c * CHC + r, pl.ds(32, L)]
            for s in range(D // L):
                sl = pl.ds(s * L, L)
                o_v[b, r, sl] = w0b * r0_v[b, r, sl] + w1b * r1_v[b, r, sl]
            return carry

        lax.fori_loop(0, CHC, tok, 0)
        store(c, b).start()
    for c in (NCC - 2, NCC - 1):
        store(c, c % 2).wait()


def kernel(hidden_states, Wg, Wgu, Wd):
    x = hidden_states

    meta, blk, slots_a = pl.pallas_call(
        _router_body,
        out_shape=[
            jax.ShapeDtypeStruct((T, 128), jnp.float32),
            jax.ShapeDtypeStruct((96, 1), jnp.int32),
            jax.ShapeDtypeStruct((2 * T, 1), jnp.int32),
        ],
    )(x, Wg)

    slots2 = slots_a.reshape(2 * T // CH, CH)
    slots3 = slots_a.reshape(2 * T // CHC, CHC)
    prefetch = blk

    mesh = plsc.VectorSubcoreMesh(core_axis_name="c", subcore_axis_name="s")

    xs = pl.kernel(
        _dispatch_body,
        out_type=jax.ShapeDtypeStruct((S, D), jnp.float32),
        mesh=mesh,
        scratch_types=[
            pltpu.VMEM((2 * NCH, CH), jnp.int32),
            pltpu.VMEM((2, CH, D), jnp.float32),
            pltpu.SemaphoreType.DMA((2,)),
            pltpu.SemaphoreType.DMA((2,)),
        ],
    )(x, slots2)

    grid_spec = pltpu.PrefetchScalarGridSpec(
        num_scalar_prefetch=1,
        grid=(G,),
        in_specs=[
            pl.BlockSpec((R, D),
                         lambda g, pref: (jnp.minimum(g, pref[G, 0] - 1), 0)),
            pl.BlockSpec(memory_space=pl.ANY),
            pl.BlockSpec(memory_space=pl.ANY),
        ],
        out_specs=pl.BlockSpec(
            (R, D), lambda g, pref: (jnp.minimum(g, pref[G, 0] - 1), 0)),
        scratch_shapes=[
            pltpu.VMEM((3, 2 * F, D), jnp.float32),
            pltpu.VMEM((3, D, F), jnp.float32),
            pltpu.SemaphoreType.DMA((3,)),
            pltpu.SemaphoreType.DMA((3,)),
        ],
    )
    ys = pl.pallas_call(
        _ffn_body,
        grid_spec=grid_spec,
        out_shape=jax.ShapeDtypeStruct((S, D), jnp.float32),
        compiler_params=pltpu.CompilerParams(
            dimension_semantics=("arbitrary",)),
    )(prefetch, xs, Wgu, Wd)

    out = pl.kernel(
        _combine_body,
        out_type=jax.ShapeDtypeStruct((T, D), jnp.float32),
        mesh=mesh,
        scratch_types=[
            pltpu.VMEM((2 * NCC, CHC), jnp.int32),
            pltpu.VMEM((TOK_W, 128), jnp.float32),
            pltpu.VMEM((2, CHC, D), jnp.float32),
            pltpu.VMEM((2, CHC, D), jnp.float32),
            pltpu.VMEM((2, CHC, D), jnp.float32),
            pltpu.SemaphoreType.DMA((2,)),
            pltpu.SemaphoreType.DMA((2,)),
            pltpu.SemaphoreType.DMA((2,)),
        ],
    )(ys, slots3, meta)
    return out


# speedup vs baseline: 1.0422x; 1.0034x over previous
import functools

import jax
import jax.numpy as jnp
from jax import lax
from jax.experimental import pallas as pl
from jax.experimental.pallas import tpu as pltpu
from jax.experimental.pallas import tpu_sc as plsc

E = 16
D = 1024
F = 768
T = 4096
R = 768
G = (2 * T + R - 1) // R + E
S = G * R
C = 512

NC = 2
NS = 16
NW = NC * NS
TOK_W = T // NW
CH = 32


def _router_body(x_ref, wg_ref, meta_ref, blk_ref, slots_ref):
    x = x_ref[...]
    wg = wg_ref[...]
    logits = lax.dot_general(x, wg, (((1,), (1,)), ((), ())),
                             preferred_element_type=jnp.float32)
    lane = lax.broadcasted_iota(jnp.int32, (T, E), 1)
    m1 = jnp.max(logits, axis=1, keepdims=True)
    e0 = jnp.min(jnp.where(logits == m1, lane, E), axis=1, keepdims=True)
    logits2 = jnp.where(lane == e0, -jnp.inf, logits)
    m2 = jnp.max(logits2, axis=1, keepdims=True)
    e1 = jnp.min(jnp.where(logits2 == m2, lane, E), axis=1, keepdims=True)
    w0 = 1.0 / (1.0 + jnp.exp(m2 - m1))
    w1 = 1.0 - w0

    oh0 = (lane == e0).astype(jnp.float32)
    oh1 = (lane == e1).astype(jnp.float32)

    a = jnp.concatenate([oh0, oh1], axis=0)
    k = 1
    while k < 2 * T:
        a = a + jnp.concatenate(
            [jnp.zeros((k, E), jnp.float32), a[:2 * T - k, :]], axis=0)
        k *= 2
    counts = a[2 * T - 1:2 * T, :]
    rank0 = a[:T, :]
    rank1 = a[T:, :]

    nblk = jnp.ceil(counts / R)
    er = lax.broadcasted_iota(jnp.int32, (E, E), 0)
    ec = lax.broadcasted_iota(jnp.int32, (E, E), 1)
    m_lt = (er < ec).astype(jnp.float32)
    m_le = (er <= ec).astype(jnp.float32)
    pad_base = lax.dot_general(nblk, m_lt, (((1,), (0,)), ((), ())),
                               preferred_element_type=jnp.float32) * R
    cum_incl = lax.dot_general(nblk, m_le, (((1,), (0,)), ((), ())),
                               preferred_element_type=jnp.float32)
    total_blocks = cum_incl[:, E - 1:E]

    slot0 = jnp.sum(oh0 * (rank0 - 1.0 + pad_base), axis=1, keepdims=True)
    slot1 = jnp.sum(oh1 * (rank1 - 1.0 + pad_base), axis=1, keepdims=True)
    slots_ref[...] = jnp.concatenate([slot0, slot1],
                                     axis=0).astype(jnp.int32)

    lane128 = lax.broadcasted_iota(jnp.int32, (T, 128), 1)
    meta = jnp.where(lane128 == 0, slot0,
                     jnp.where(lane128 == 1, slot1,
                               jnp.where(lane128 < 32, w0, w1)))
    meta_ref[...] = meta

    gidx = lax.broadcasted_iota(jnp.int32, (96, E), 0).astype(jnp.float32)
    act = (counts > 0.0).astype(jnp.float32)
    cum_excl = cum_incl - nblk
    arank = lax.dot_general(act, m_lt, (((1,), (0,)), ((), ())),
                            preferred_element_type=jnp.float32)
    bexp = jnp.sum((gidx >= cum_incl).astype(jnp.float32), axis=1,
                   keepdims=True)
    bexp = jnp.minimum(bexp, float(E - 1))
    is_first = (gidx == cum_excl) * act
    chg = jnp.sum(is_first, axis=1, keepdims=True)
    run_idx = jnp.sum((gidx >= cum_excl) * act, axis=1,
                      keepdims=True) - 1.0
    slot = run_idx - 3.0 * jnp.floor(run_idx / 3.0)
    eids = lax.broadcasted_iota(jnp.int32, (96, E), 1).astype(jnp.float32)
    sel1 = (arank == run_idx + 1.0) * act
    er1 = jnp.sum(sel1 * eids, axis=1, keepdims=True)
    v1 = jnp.minimum(jnp.sum(sel1, axis=1, keepdims=True), 1.0)
    sel2 = (arank == run_idx + 2.0) * act
    er2 = jnp.sum(sel2 * eids, axis=1, keepdims=True)
    v2 = jnp.minimum(jnp.sum(sel2, axis=1, keepdims=True), 1.0)
    packed = (bexp + slot * 32.0 + chg * 128.0 + er1 * 256.0
              + v1 * 8192.0 + er2 * 16384.0 + v2 * 524288.0)
    row = lax.broadcasted_iota(jnp.int32, (96, 1), 0)
    vals = jnp.where(row == G, total_blocks, packed)
    blk_ref[...] = jnp.broadcast_to(vals, (96, 128)).astype(jnp.int32)


NCH = TOK_W // CH


def _dispatch_body(x_hbm, slots2_hbm, xs_hbm, idx_v, rows_v, semL, semS):
    wid = lax.axis_index("s") * NC + lax.axis_index("c")
    base = wid * TOK_W
    rb = wid * NCH

    pltpu.sync_copy(slots2_hbm.at[pl.ds(rb, NCH)], idx_v.at[pl.ds(0, NCH)])
    pltpu.sync_copy(slots2_hbm.at[pl.ds(T // CH + rb, NCH)],
                    idx_v.at[pl.ds(NCH, NCH)])

    def load(c, b):
        return pltpu.make_async_copy(x_hbm.at[pl.ds(base + c * CH, CH)],
                                     rows_v.at[b], semL.at[b])

    def scat(c, b, k):
        return pltpu.make_async_copy(rows_v.at[b],
                                     xs_hbm.at[idx_v.at[k * NCH + c]],
                                     semS.at[b])

    load(0, 0).start()
    for c in range(NCH):
        b = c % 2
        load(c, b).wait()
        if c + 1 < NCH:
            if c >= 1:
                scat(c - 1, 1 - b, 0).wait()
                scat(c - 1, 1 - b, 1).wait()
            load(c + 1, 1 - b).start()
        scat(c, b, 0).start()
        scat(c, b, 1).start()
    for c in (NCH - 2, NCH - 1):
        scat(c, c % 2, 0).wait()
        scat(c, c % 2, 1).wait()


def _ffn_body(info_ref, xs_ref, wgu_hbm, wd_hbm, ys_ref,
              wgu_buf, wd_buf, sem_gu, sem_d):
    g = pl.program_id(0)
    nb = info_ref[G]
    info = info_ref[g]
    eid = lax.rem(info, 32)
    slot = lax.rem(info // 32, 4)
    chg = lax.rem(info // 128, 2)
    er1 = lax.rem(info // 256, 32)
    v1 = lax.rem(info // 8192, 2)
    er2 = lax.rem(info // 16384, 32)
    v2 = lax.rem(info // 524288, 2)

    def start_fetch(e, s):
        pltpu.make_async_copy(wgu_hbm.at[e], wgu_buf.at[s],
                              sem_gu.at[s]).start()
        pltpu.make_async_copy(wd_hbm.at[e], wd_buf.at[s],
                              sem_d.at[s]).start()

    @pl.when(g == 0)
    def _():
        start_fetch(eid, slot)

        @pl.when(v1 == 1)
        def _():
            start_fetch(er1, lax.rem(slot + 1, 3))

    @pl.when((g < nb) & (chg == 1))
    def _():
        @pl.when(v2 == 1)
        def _():
            start_fetch(er2, lax.rem(slot + 2, 3))

        pltpu.make_async_copy(wgu_hbm.at[eid], wgu_buf.at[slot],
                              sem_gu.at[slot]).wait()
        pltpu.make_async_copy(wd_hbm.at[eid], wd_buf.at[slot],
                              sem_d.at[slot]).wait()

    @pl.when(g < nb)
    def _():
        x = xs_ref[...]
        wgu = wgu_buf[slot]
        gu = lax.dot_general(x, wgu, (((1,), (1,)), ((), ())),
                             preferred_element_type=jnp.float32)
        gate = gu[:, :F]
        up = gu[:, F:]
        h = gate * (1.0 / (1.0 + jnp.exp(-gate))) * up
        wd = wd_buf[slot]
        ys_ref[...] = lax.dot_general(h, wd, (((1,), (1,)), ((), ())),
                                      preferred_element_type=jnp.float32)


CHC = 16
NCC = TOK_W // CHC


def _combine_body(ys_hbm, slots3_hbm, meta_hbm, out_hbm, idx_v, wr_v,
                  r0_v, r1_v, o_v, semG0, semG1, semO):
    wid = lax.axis_index("s") * NC + lax.axis_index("c")
    base = wid * TOK_W
    rb = wid * NCC
    L = 16

    pltpu.sync_copy(slots3_hbm.at[pl.ds(rb, NCC)], idx_v.at[pl.ds(0, NCC)])
    pltpu.sync_copy(slots3_hbm.at[pl.ds(T // CHC + rb, NCC)],
                    idx_v.at[pl.ds(NCC, NCC)])
    pltpu.sync_copy(meta_hbm.at[pl.ds(base, TOK_W)], wr_v)

    def gath(c, b, k, dst, sem):
        return pltpu.make_async_copy(ys_hbm.at[idx_v.at[k * NCC + c]],
                                     dst.at[b], sem.at[b])

    def store(c, b):
        return pltpu.make_async_copy(o_v.at[b],
                                     out_hbm.at[pl.ds(base + c * CHC, CHC)],
                                     semO.at[b])

    gath(0, 0, 0, r0_v, semG0).start()
    gath(0, 0, 1, r1_v, semG1).start()
    for c in range(NCC):
        b = c % 2
        if c + 1 < NCC:
            gath(c + 1, 1 - b, 0, r0_v, semG0).start()
            gath(c + 1, 1 - b, 1, r1_v, semG1).start()
        gath(c, b, 0, r0_v, semG0).wait()
        gath(c, b, 1, r1_v, semG1).wait()
        if c >= 2:
            store(c - 2, b).wait()

        def tok(r, carry):
            w0b = wr_v[c * CHC + r, pl.ds(16, L)]
            w1b = wr_v[c * CHC + r, pl.ds(32, L)]
            for s in range(D // L):
                sl = pl.ds(s * L, L)
                o_v[b, r, sl] = w0b * r0_v[b, r, sl] + w1b * r1_v[b, r, sl]
            return carry

        lax.fori_loop(0, CHC, tok, 0)
        store(c, b).start()
    for c in (NCC - 2, NCC - 1):
        store(c, c % 2).wait()


def kernel(hidden_states, Wg, Wgu, Wd):
    x = hidden_states

    meta, blk, slots_a = pl.pallas_call(
        _router_body,
        out_shape=[
            jax.ShapeDtypeStruct((T, 128), jnp.float32),
            jax.ShapeDtypeStruct((96, 128), jnp.int32),
            jax.ShapeDtypeStruct((2 * T, 1), jnp.int32),
        ],
    )(x, Wg)

    slots2 = slots_a.reshape(2 * T // CH, CH)
    slots3 = slots_a.reshape(2 * T // CHC, CHC)
    prefetch = blk[:G + 1, 0]

    mesh = plsc.VectorSubcoreMesh(core_axis_name="c", subcore_axis_name="s")

    xs = pl.kernel(
        _dispatch_body,
        out_type=jax.ShapeDtypeStruct((S, D), jnp.float32),
        mesh=mesh,
        scratch_types=[
            pltpu.VMEM((2 * NCH, CH), jnp.int32),
            pltpu.VMEM((2, CH, D), jnp.float32),
            pltpu.SemaphoreType.DMA((2,)),
            pltpu.SemaphoreType.DMA((2,)),
        ],
    )(x, slots2)

    grid_spec = pltpu.PrefetchScalarGridSpec(
        num_scalar_prefetch=1,
        grid=(G,),
        in_specs=[
            pl.BlockSpec((R, D),
                         lambda g, pref: (jnp.minimum(g, pref[G] - 1), 0)),
            pl.BlockSpec(memory_space=pl.ANY),
            pl.BlockSpec(memory_space=pl.ANY),
        ],
        out_specs=pl.BlockSpec(
            (R, D), lambda g, pref: (jnp.minimum(g, pref[G] - 1), 0)),
        scratch_shapes=[
            pltpu.VMEM((3, 2 * F, D), jnp.float32),
            pltpu.VMEM((3, D, F), jnp.float32),
            pltpu.SemaphoreType.DMA((3,)),
            pltpu.SemaphoreType.DMA((3,)),
        ],
    )
    ys = pl.pallas_call(
        _ffn_body,
        grid_spec=grid_spec,
        out_shape=jax.ShapeDtypeStruct((S, D), jnp.float32),
        compiler_params=pltpu.CompilerParams(
            dimension_semantics=("arbitrary",)),
    )(prefetch, xs, Wgu, Wd)

    out = pl.kernel(
        _combine_body,
        out_type=jax.ShapeDtypeStruct((T, D), jnp.float32),
        mesh=mesh,
        scratch_types=[
            pltpu.VMEM((2 * NCC, CHC), jnp.int32),
            pltpu.VMEM((TOK_W, 128), jnp.float32),
            pltpu.VMEM((2, CHC, D), jnp.float32),
            pltpu.VMEM((2, CHC, D), jnp.float32),
            pltpu.VMEM((2, CHC, D), jnp.float32),
            pltpu.SemaphoreType.DMA((2,)),
            pltpu.SemaphoreType.DMA((2,)),
            pltpu.SemaphoreType.DMA((2,)),
        ],
    )(ys, slots3, meta)
    return out


# speedup vs baseline: 1.0422x; 1.0000x over previous
import jax
import jax.numpy as jnp
from jax import lax
from jax.experimental import pallas as pl
from jax.experimental.pallas import tpu as pltpu
from jax.experimental.pallas import tpu_sc as plsc

E = 16
D = 1024
F = 768
T = 4096
R = 768
G = (2 * T + R - 1) // R + E
S = G * R

NC = 2
NS = 16
NW = NC * NS
TOK_W = T // NW
CH = 32


def _router_body(x_ref, wg_ref, meta_ref, blk_ref, slots_ref):
    x = x_ref[...]
    wg = wg_ref[...]
    logits = lax.dot_general(x, wg, (((1,), (1,)), ((), ())),
                             preferred_element_type=jnp.float32)
    lane = lax.broadcasted_iota(jnp.int32, (T, E), 1)
    m1 = jnp.max(logits, axis=1, keepdims=True)
    e0 = jnp.min(jnp.where(logits == m1, lane, E), axis=1, keepdims=True)
    logits2 = jnp.where(lane == e0, -jnp.inf, logits)
    m2 = jnp.max(logits2, axis=1, keepdims=True)
    e1 = jnp.min(jnp.where(logits2 == m2, lane, E), axis=1, keepdims=True)
    w0 = 1.0 / (1.0 + jnp.exp(m2 - m1))
    w1 = 1.0 - w0

    oh0 = (lane == e0).astype(jnp.float32)
    oh1 = (lane == e1).astype(jnp.float32)

    a = jnp.concatenate([oh0, oh1], axis=0)
    k = 1
    while k < 2 * T:
        a = a + jnp.concatenate(
            [jnp.zeros((k, E), jnp.float32), a[:2 * T - k, :]], axis=0)
        k *= 2
    counts = a[2 * T - 1:2 * T, :]
    rank0 = a[:T, :]
    rank1 = a[T:, :]

    nblk = jnp.ceil(counts / R)
    er = lax.broadcasted_iota(jnp.int32, (E, E), 0)
    ec = lax.broadcasted_iota(jnp.int32, (E, E), 1)
    m_lt = (er < ec).astype(jnp.float32)
    m_le = (er <= ec).astype(jnp.float32)
    pad_base = lax.dot_general(nblk, m_lt, (((1,), (0,)), ((), ())),
                               preferred_element_type=jnp.float32) * R
    cum_incl = lax.dot_general(nblk, m_le, (((1,), (0,)), ((), ())),
                               preferred_element_type=jnp.float32)
    total_blocks = cum_incl[:, E - 1:E]

    slot0 = jnp.sum(oh0 * (rank0 - 1.0 + pad_base), axis=1, keepdims=True)
    slot1 = jnp.sum(oh1 * (rank1 - 1.0 + pad_base), axis=1, keepdims=True)
    slots_ref[...] = jnp.concatenate([slot0, slot1],
                                     axis=0).astype(jnp.int32)

    lane128 = lax.broadcasted_iota(jnp.int32, (T, 128), 1)
    meta = jnp.where(lane128 == 0, slot0,
                     jnp.where(lane128 == 1, slot1,
                               jnp.where(lane128 < 32, w0, w1)))
    meta_ref[...] = meta

    gidx = lax.broadcasted_iota(jnp.int32, (96, E), 0).astype(jnp.float32)
    act = (counts > 0.0).astype(jnp.float32)
    cum_excl = cum_incl - nblk
    arank = lax.dot_general(act, m_lt, (((1,), (0,)), ((), ())),
                            preferred_element_type=jnp.float32)
    bexp = jnp.sum((gidx >= cum_incl).astype(jnp.float32), axis=1,
                   keepdims=True)
    bexp = jnp.minimum(bexp, float(E - 1))
    is_first = (gidx == cum_excl) * act
    chg = jnp.sum(is_first, axis=1, keepdims=True)
    run_idx = jnp.sum((gidx >= cum_excl) * act, axis=1,
                      keepdims=True) - 1.0
    slot = run_idx - 3.0 * jnp.floor(run_idx / 3.0)
    eids = lax.broadcasted_iota(jnp.int32, (96, E), 1).astype(jnp.float32)
    sel1 = (arank == run_idx + 1.0) * act
    er1 = jnp.sum(sel1 * eids, axis=1, keepdims=True)
    v1 = jnp.minimum(jnp.sum(sel1, axis=1, keepdims=True), 1.0)
    sel2 = (arank == run_idx + 2.0) * act
    er2 = jnp.sum(sel2 * eids, axis=1, keepdims=True)
    v2 = jnp.minimum(jnp.sum(sel2, axis=1, keepdims=True), 1.0)
    packed = (bexp + slot * 32.0 + chg * 128.0 + er1 * 256.0
              + v1 * 8192.0 + er2 * 16384.0 + v2 * 524288.0)
    row = lax.broadcasted_iota(jnp.int32, (96, 1), 0)
    vals = jnp.where(row == G, total_blocks, packed)
    blk_ref[...] = jnp.broadcast_to(vals, (96, 128)).astype(jnp.int32)


NCH = TOK_W // CH


def _dispatch_body(x_hbm, slots2_hbm, xs_hbm, idx_v, rows_v, semL, semS):
    wid = lax.axis_index("s") * NC + lax.axis_index("c")
    base = wid * TOK_W
    rb = wid * NCH

    pltpu.sync_copy(slots2_hbm.at[pl.ds(rb, NCH)], idx_v.at[pl.ds(0, NCH)])
    pltpu.sync_copy(slots2_hbm.at[pl.ds(T // CH + rb, NCH)],
                    idx_v.at[pl.ds(NCH, NCH)])

    def load(c, b):
        return pltpu.make_async_copy(x_hbm.at[pl.ds(base + c * CH, CH)],
                                     rows_v.at[b], semL.at[b])

    def scat(c, b, k):
        return pltpu.make_async_copy(rows_v.at[b],
                                     xs_hbm.at[idx_v.at[k * NCH + c]],
                                     semS.at[b])

    load(0, 0).start()
    for c in range(NCH):
        b = c % 2
        load(c, b).wait()
        if c + 1 < NCH:
            if c >= 1:
                scat(c - 1, 1 - b, 0).wait()
                scat(c - 1, 1 - b, 1).wait()
            load(c + 1, 1 - b).start()
        scat(c, b, 0).start()
        scat(c, b, 1).start()
    for c in (NCH - 2, NCH - 1):
        scat(c, c % 2, 0).wait()
        scat(c, c % 2, 1).wait()


def _ffn_body(info_ref, xs_ref, wgu_hbm, wd_hbm, ys_ref,
              wgu_buf, wd_buf, sem_gu, sem_d):
    g = pl.program_id(0)
    nb = info_ref[G]
    info = info_ref[g]
    eid = lax.rem(info, 32)
    slot = lax.rem(info // 32, 4)
    chg = lax.rem(info // 128, 2)
    er1 = lax.rem(info // 256, 32)
    v1 = lax.rem(info // 8192, 2)
    er2 = lax.rem(info // 16384, 32)
    v2 = lax.rem(info // 524288, 2)

    def start_fetch(e, s):
        pltpu.make_async_copy(wgu_hbm.at[e], wgu_buf.at[s],
                              sem_gu.at[s]).start()
        pltpu.make_async_copy(wd_hbm.at[e], wd_buf.at[s],
                              sem_d.at[s]).start()

    @pl.when(g == 0)
    def _():
        start_fetch(eid, slot)

        @pl.when(v1 == 1)
        def _():
            start_fetch(er1, lax.rem(slot + 1, 3))

    @pl.when((g < nb) & (chg == 1))
    def _():
        @pl.when(v2 == 1)
        def _():
            start_fetch(er2, lax.rem(slot + 2, 3))

        pltpu.make_async_copy(wgu_hbm.at[eid], wgu_buf.at[slot],
                              sem_gu.at[slot]).wait()
        pltpu.make_async_copy(wd_hbm.at[eid], wd_buf.at[slot],
                              sem_d.at[slot]).wait()

    @pl.when(g < nb)
    def _():
        x = xs_ref[...]
        wgu = wgu_buf[slot]
        gu = lax.dot_general(x, wgu, (((1,), (1,)), ((), ())),
                             preferred_element_type=jnp.float32)
        gate = gu[:, :F]
        up = gu[:, F:]
        h = gate * (1.0 / (1.0 + jnp.exp(-gate))) * up
        wd = wd_buf[slot]
        ys_ref[...] = lax.dot_general(h, wd, (((1,), (1,)), ((), ())),
                                      preferred_element_type=jnp.float32)


CHC = 16
NCC = TOK_W // CHC


def _combine_body(ys_hbm, slots3_hbm, meta_hbm, out_hbm, idx_v, wr_v,
                  r0_v, r1_v, o_v, semG0, semG1, semO):
    wid = lax.axis_index("s") * NC + lax.axis_index("c")
    base = wid * TOK_W
    rb = wid * NCC
    L = 16

    pltpu.sync_copy(slots3_hbm.at[pl.ds(rb, NCC)], idx_v.at[pl.ds(0, NCC)])
    pltpu.sync_copy(slots3_hbm.at[pl.ds(T // CHC + rb, NCC)],
                    idx_v.at[pl.ds(NCC, NCC)])
    pltpu.sync_copy(meta_hbm.at[pl.ds(base, TOK_W)], wr_v)

    def gath(c, b, k, dst, sem):
        return pltpu.make_async_copy(ys_hbm.at[idx_v.at[k * NCC + c]],
                                     dst.at[b], sem.at[b])

    def store(c, b):
        return pltpu.make_async_copy(o_v.at[b],
                                     out_hbm.at[pl.ds(base + c * CHC, CHC)],
                                     semO.at[b])

    gath(0, 0, 0, r0_v, semG0).start()
    gath(0, 0, 1, r1_v, semG1).start()
    for c in range(NCC):
        b = c % 2
        if c + 1 < NCC:
            gath(c + 1, 1 - b, 0, r0_v, semG0).start()
            gath(c + 1, 1 - b, 1, r1_v, semG1).start()
        gath(c, b, 0, r0_v, semG0).wait()
        gath(c, b, 1, r1_v, semG1).wait()
        if c >= 2:
            store(c - 2, b).wait()

        def tok(r, carry):
            w0b = wr_v[c * CHC + r, pl.ds(16, L)]
            w1b = wr_v[c * CHC + r, pl.ds(32, L)]
            for s in range(D // L):
                sl = pl.ds(s * L, L)
                o_v[b, r, sl] = w0b * r0_v[b, r, sl] + w1b * r1_v[b, r, sl]
            return carry

        lax.fori_loop(0, CHC, tok, 0)
        store(c, b).start()
    for c in (NCC - 2, NCC - 1):
        store(c, c % 2).wait()


def kernel(hidden_states, Wg, Wgu, Wd):
    x = hidden_states

    meta, blk, slots_a = pl.pallas_call(
        _router_body,
        out_shape=[
            jax.ShapeDtypeStruct((T, 128), jnp.float32),
            jax.ShapeDtypeStruct((96, 128), jnp.int32),
            jax.ShapeDtypeStruct((2 * T, 1), jnp.int32),
        ],
    )(x, Wg)

    slots2 = slots_a.reshape(2 * T // CH, CH)
    slots3 = slots_a.reshape(2 * T // CHC, CHC)
    prefetch = blk[:G + 1, 0]

    mesh = plsc.VectorSubcoreMesh(core_axis_name="c", subcore_axis_name="s")

    xs = pl.kernel(
        _dispatch_body,
        out_type=jax.ShapeDtypeStruct((S, D), jnp.float32),
        mesh=mesh,
        scratch_types=[
            pltpu.VMEM((2 * NCH, CH), jnp.int32),
            pltpu.VMEM((2, CH, D), jnp.float32),
            pltpu.SemaphoreType.DMA((2,)),
            pltpu.SemaphoreType.DMA((2,)),
        ],
    )(x, slots2)

    grid_spec = pltpu.PrefetchScalarGridSpec(
        num_scalar_prefetch=1,
        grid=(G,),
        in_specs=[
            pl.BlockSpec((R, D),
                         lambda g, pref: (jnp.minimum(g, pref[G] - 1), 0)),
            pl.BlockSpec(memory_space=pl.ANY),
            pl.BlockSpec(memory_space=pl.ANY),
        ],
        out_specs=pl.BlockSpec(
            (R, D), lambda g, pref: (jnp.minimum(g, pref[G] - 1), 0)),
        scratch_shapes=[
            pltpu.VMEM((3, 2 * F, D), jnp.float32),
            pltpu.VMEM((3, D, F), jnp.float32),
            pltpu.SemaphoreType.DMA((3,)),
            pltpu.SemaphoreType.DMA((3,)),
        ],
    )
    ys = pl.pallas_call(
        _ffn_body,
        grid_spec=grid_spec,
        out_shape=jax.ShapeDtypeStruct((S, D), jnp.float32),
        compiler_params=pltpu.CompilerParams(
            dimension_semantics=("arbitrary",)),
    )(prefetch, xs, Wgu, Wd)

    out = pl.kernel(
        _combine_body,
        out_type=jax.ShapeDtypeStruct((T, D), jnp.float32),
        mesh=mesh,
        scratch_types=[
            pltpu.VMEM((2 * NCC, CHC), jnp.int32),
            pltpu.VMEM((TOK_W, 128), jnp.float32),
            pltpu.VMEM((2, CHC, D), jnp.float32),
            pltpu.VMEM((2, CHC, D), jnp.float32),
            pltpu.VMEM((2, CHC, D), jnp.float32),
            pltpu.SemaphoreType.DMA((2,)),
            pltpu.SemaphoreType.DMA((2,)),
            pltpu.SemaphoreType.DMA((2,)),
        ],
    )(ys, slots3, meta)
    return out
